# baseline (speedup 1.0000x reference)
import functools

import jax
import jax.numpy as jnp
from jax import lax
from jax.experimental import pallas as pl
from jax.experimental.pallas import tpu as pltpu
from jax.experimental.pallas import tpu_sc as plsc

N = 10000
E = 320000
D = 128
H = 128
C = 40
CP = 64

NW = 32
K = 128
NCHUNK = 80
GK = 8
NG = NCHUNK // GK
EP = NW * NCHUNK * K
EPW = EP // NW
NPAD = 10240
RPT = NPAD // 16
ZR = K


def _agg_body(p_hbm, src_hbm, dst_hbm, zero_hbm, out_hbm,
              accum, src_v, dst_v, buf0, buf1, gsem0, gsem1, isem):
    c = lax.axis_index("c")
    s = lax.axis_index("s")
    w = c * 16 + s
    base = s * RPT
    bufs = (buf0, buf1)
    sems = (gsem0, gsem1)

    pltpu.sync_copy(zero_hbm, buf0)
    for i in range(RPT // ZR):
        pltpu.async_copy(buf0, accum.at[pl.ds(base + i * ZR, ZR)], isem)
    pltpu.sync_copy(src_hbm.at[w, pl.ds(0, GK)], src_v.at[0])
    pltpu.sync_copy(dst_hbm.at[w, pl.ds(0, GK)], dst_v.at[0])
    for i in range(RPT // ZR):
        pltpu.make_async_copy(buf0, accum.at[pl.ds(base + i * ZR, ZR)],
                              isem).wait()
    plsc.subcore_barrier()

    pltpu.async_copy(p_hbm.at[src_v.at[0, 0]], buf0, gsem0)

    def group(g, carry):
        p = lax.rem(g, 2)
        q = 1 - p
        more = g + 1 < NG

        @pl.when(more)
        def _prefetch():
            pltpu.async_copy(src_hbm.at[w, pl.ds((g + 1) * GK, GK)],
                             src_v.at[q], isem)
            pltpu.async_copy(dst_hbm.at[w, pl.ds((g + 1) * GK, GK)],
                             dst_v.at[q], isem)

        for j in range(GK):
            b = j % 2
            nb = 1 - b
            if j + 1 < GK:
                pltpu.async_copy(p_hbm.at[src_v.at[p, j + 1]],
                                 bufs[nb], sems[nb])
            else:
                @pl.when(more)
                def _next_group():
                    pltpu.make_async_copy(
                        src_hbm.at[w, pl.ds((g + 1) * GK, GK)],
                        src_v.at[q], isem).wait()
                    pltpu.make_async_copy(
                        dst_hbm.at[w, pl.ds((g + 1) * GK, GK)],
                        dst_v.at[q], isem).wait()
                    pltpu.async_copy(p_hbm.at[src_v.at[q, 0]],
                                     bufs[nb], sems[nb])
            pltpu.make_async_copy(p_hbm.at[src_v.at[p, j]],
                                  bufs[b], sems[b]).wait()
            pltpu.sync_copy(bufs[b], accum.at[dst_v.at[p, j]], add=True)
        return carry

    lax.fori_loop(0, NG, group, 0)

    plsc.subcore_barrier()
    for i in range(RPT // ZR):
        pltpu.async_copy(accum.at[pl.ds(base + i * ZR, ZR)],
                         out_hbm.at[pl.ds(c * NPAD + base + i * ZR, ZR)],
                         isem)
    for i in range(RPT // ZR):
        pltpu.make_async_copy(accum.at[pl.ds(base + i * ZR, ZR)],
                              out_hbm.at[pl.ds(c * NPAD + base + i * ZR, ZR)],
                              isem).wait()


@functools.cache
def _make_agg(dw: int):
    mesh = plsc.VectorSubcoreMesh(core_axis_name="c", subcore_axis_name="s")
    params = None
    if dw != 128:
        params = pltpu.CompilerParams(use_tc_tiling_on_sc=False)
    return pl.kernel(
        _agg_body,
        out_type=jax.ShapeDtypeStruct((2 * NPAD, dw), jnp.float32),
        mesh=mesh,
        compiler_params=params,
        scratch_types=[
            pltpu.VMEM_SHARED((NPAD, dw), jnp.float32),
            pltpu.VMEM((2, GK, K), jnp.int32),
            pltpu.VMEM((2, GK, K), jnp.int32),
            pltpu.VMEM((K, dw), jnp.float32),
            pltpu.VMEM((K, dw), jnp.float32),
            pltpu.SemaphoreType.DMA,
            pltpu.SemaphoreType.DMA,
            pltpu.SemaphoreType.DMA,
        ],
    )


_RB = 1000
_RBP = 1024
_NBP = NPAD // _RBP


def _proj_body(x_ref, w_ref, o_ref):
    o_ref[...] = jnp.dot(x_ref[...], w_ref[...],
                         preferred_element_type=jnp.float32)


def _proj(x, w):
    n, d = x.shape
    dout = w.shape[1]
    return pl.pallas_call(
        _proj_body,
        grid=(n // _RB,),
        in_specs=[pl.BlockSpec((_RB, d), lambda i: (i, 0)),
                  pl.BlockSpec((d, dout), lambda i: (0, 0))],
        out_specs=pl.BlockSpec((_RB, dout), lambda i: (i, 0)),
        out_shape=jax.ShapeDtypeStruct((n, dout), jnp.float32),
    )(x, w)


def _fuse_body(ma_ref, mb_ref, n_ref, b_ref, w_ref, o_ref):
    m = ma_ref[...] + mb_ref[...]
    h = jnp.maximum(m * n_ref[...] + b_ref[...][None, :], 0.0)
    o_ref[...] = jnp.dot(h, w_ref[...], preferred_element_type=jnp.float32)


def _fuse(mm, normp, b, w):
    d = mm.shape[1]
    dout = w.shape[1]
    return pl.pallas_call(
        _fuse_body,
        grid=(_NBP,),
        in_specs=[pl.BlockSpec((_RBP, d), lambda i: (i, 0)),
                  pl.BlockSpec((_RBP, d), lambda i: (_NBP + i, 0)),
                  pl.BlockSpec((_RBP, 1), lambda i: (i, 0)),
                  pl.BlockSpec((d,), lambda i: (0,)),
                  pl.BlockSpec((d, dout), lambda i: (0, 0))],
        out_specs=pl.BlockSpec((_RBP, dout), lambda i: (i, 0)),
        out_shape=jax.ShapeDtypeStruct((NPAD, dout), jnp.float32),
    )(mm, mm, normp, b, w)


def _final_body(ma_ref, mb_ref, n_ref, b_ref, o_ref):
    m = ma_ref[...] + mb_ref[...]
    o_ref[...] = m * n_ref[...] + b_ref[...][None, :]


def _final(mm, normp, b):
    d = mm.shape[1]
    return pl.pallas_call(
        _final_body,
        grid=(_NBP,),
        in_specs=[pl.BlockSpec((_RBP, d), lambda i: (i, 0)),
                  pl.BlockSpec((_RBP, d), lambda i: (_NBP + i, 0)),
                  pl.BlockSpec((_RBP, 1), lambda i: (i, 0)),
                  pl.BlockSpec((d,), lambda i: (0,))],
        out_specs=pl.BlockSpec((_RBP, d), lambda i: (i, 0)),
        out_shape=jax.ShapeDtypeStruct((NPAD, d), jnp.float32),
    )(mm, mm, normp, b)


def kernel(x, edge_index, norm, W0, b0, W1, b1, W2, b2):
    epw_real = E // NW
    pade = EPW - epw_real
    pad_s = jnp.broadcast_to(
        (jnp.arange(pade, dtype=jnp.int32) * 41) % N, (NW, pade))
    pad_d = jnp.broadcast_to(
        N + jnp.arange(pade, dtype=jnp.int32) % (NPAD - N), (NW, pade))
    src = jnp.concatenate(
        [edge_index[0].reshape(NW, epw_real), pad_s], axis=1
    ).reshape(NW, NCHUNK, K)
    dst = jnp.concatenate(
        [edge_index[1].reshape(NW, epw_real), pad_d], axis=1
    ).reshape(NW, NCHUNK, K)
    normp = jnp.pad(norm.reshape(N, 1), ((0, NPAD - N), (0, 0)))
    zero128 = jnp.zeros((ZR, D), jnp.float32)
    zero64 = jnp.zeros((ZR, CP), jnp.float32)
    W2p = jnp.pad(W2, ((0, 0), (0, CP - C)))
    b2p = jnp.pad(b2, (0, CP - C))

    agg_d = _make_agg(D)
    agg_c = _make_agg(CP)

    p0 = _proj(x, W0)
    m0 = agg_d(p0, src, dst, zero128)
    p1 = _fuse(m0, normp, b0, W1)
    m1 = agg_d(p1, src, dst, zero128)
    p2 = _fuse(m1, normp, b1, W2p)
    m2 = agg_c(p2, src, dst, zero64)
    out = _final(m2, normp, b2p)
    return out[:N, :C]


# speedup vs baseline: 1.0257x; 1.0257x over previous
import functools

import jax
import jax.numpy as jnp
from jax import lax
from jax.experimental import pallas as pl
from jax.experimental.pallas import tpu as pltpu
from jax.experimental.pallas import tpu_sc as plsc

N = 10000
E = 320000
D = 128
H = 128
C = 40
CP = 40

NW = 32
K = 128
NCHUNK = 80
GK = 8
NG = NCHUNK // GK
EP = NW * NCHUNK * K
EPW = EP // NW
NPAD = 10240
RPT = NPAD // 16
ZR = K


def _agg_body(p_hbm, src_hbm, dst_hbm, zero_hbm, out_hbm,
              accum, src_v, dst_v, buf0, buf1, gsem0, gsem1, isem):
    c = lax.axis_index("c")
    s = lax.axis_index("s")
    w = c * 16 + s
    base = s * RPT
    bufs = (buf0, buf1)
    sems = (gsem0, gsem1)

    pltpu.sync_copy(zero_hbm, buf0)
    for i in range(RPT // ZR):
        pltpu.async_copy(buf0, accum.at[pl.ds(base + i * ZR, ZR)], isem)
    pltpu.sync_copy(src_hbm.at[w, pl.ds(0, GK)], src_v.at[0])
    pltpu.sync_copy(dst_hbm.at[w, pl.ds(0, GK)], dst_v.at[0])
    for i in range(RPT // ZR):
        pltpu.make_async_copy(buf0, accum.at[pl.ds(base + i * ZR, ZR)],
                              isem).wait()
    plsc.subcore_barrier()

    pltpu.async_copy(p_hbm.at[src_v.at[0, 0]], buf0, gsem0)

    def group(g, carry):
        p = lax.rem(g, 2)
        q = 1 - p
        more = g + 1 < NG

        @pl.when(more)
        def _prefetch():
            pltpu.async_copy(src_hbm.at[w, pl.ds((g + 1) * GK, GK)],
                             src_v.at[q], isem)
            pltpu.async_copy(dst_hbm.at[w, pl.ds((g + 1) * GK, GK)],
                             dst_v.at[q], isem)

        for j in range(GK):
            b = j % 2
            nb = 1 - b
            if j + 1 < GK:
                pltpu.async_copy(p_hbm.at[src_v.at[p, j + 1]],
                                 bufs[nb], sems[nb])
            else:
                @pl.when(more)
                def _next_group():
                    pltpu.make_async_copy(
                        src_hbm.at[w, pl.ds((g + 1) * GK, GK)],
                        src_v.at[q], isem).wait()
                    pltpu.make_async_copy(
                        dst_hbm.at[w, pl.ds((g + 1) * GK, GK)],
                        dst_v.at[q], isem).wait()
                    pltpu.async_copy(p_hbm.at[src_v.at[q, 0]],
                                     bufs[nb], sems[nb])
            pltpu.make_async_copy(p_hbm.at[src_v.at[p, j]],
                                  bufs[b], sems[b]).wait()
            pltpu.sync_copy(bufs[b], accum.at[dst_v.at[p, j]], add=True)
        return carry

    lax.fori_loop(0, NG, group, 0)

    plsc.subcore_barrier()
    pltpu.sync_copy(accum.at[pl.ds(base, ZR)], buf0)
    for i in range(RPT // ZR):
        bo = bufs[i % 2]
        bn = bufs[(i + 1) % 2]
        if i + 1 < RPT // ZR:
            pltpu.async_copy(accum.at[pl.ds(base + (i + 1) * ZR, ZR)],
                             bn, isem)
        pltpu.sync_copy(bo, out_hbm.at[pl.ds(c * NPAD + base + i * ZR, ZR)])
        if i + 1 < RPT // ZR:
            pltpu.make_async_copy(accum.at[pl.ds(base + (i + 1) * ZR, ZR)],
                                  bn, isem).wait()


@functools.cache
def _make_agg(dw: int):
    mesh = plsc.VectorSubcoreMesh(core_axis_name="c", subcore_axis_name="s")
    params = None
    if dw != 128:
        params = pltpu.CompilerParams(use_tc_tiling_on_sc=False)
    return pl.kernel(
        _agg_body,
        out_type=jax.ShapeDtypeStruct((2 * NPAD, dw), jnp.float32),
        mesh=mesh,
        compiler_params=params,
        scratch_types=[
            pltpu.VMEM_SHARED((NPAD, dw), jnp.float32),
            pltpu.VMEM((2, GK, K), jnp.int32),
            pltpu.VMEM((2, GK, K), jnp.int32),
            pltpu.VMEM((K, dw), jnp.float32),
            pltpu.VMEM((K, dw), jnp.float32),
            pltpu.SemaphoreType.DMA,
            pltpu.SemaphoreType.DMA,
            pltpu.SemaphoreType.DMA,
        ],
    )


_RB = 1000
_RBP = 1024
_NBP = NPAD // _RBP


def _proj_body(x_ref, w_ref, o_ref):
    o_ref[...] = jnp.dot(x_ref[...], w_ref[...],
                         preferred_element_type=jnp.float32)


def _proj(x, w):
    n, d = x.shape
    dout = w.shape[1]
    return pl.pallas_call(
        _proj_body,
        grid=(n // _RB,),
        in_specs=[pl.BlockSpec((_RB, d), lambda i: (i, 0)),
                  pl.BlockSpec((d, dout), lambda i: (0, 0))],
        out_specs=pl.BlockSpec((_RB, dout), lambda i: (i, 0)),
        out_shape=jax.ShapeDtypeStruct((n, dout), jnp.float32),
    )(x, w)


def _fuse_body(ma_ref, mb_ref, n_ref, b_ref, w_ref, o_ref):
    m = ma_ref[...] + mb_ref[...]
    h = jnp.maximum(m * n_ref[...] + b_ref[...][None, :], 0.0)
    o_ref[...] = jnp.dot(h, w_ref[...], preferred_element_type=jnp.float32)


def _fuse(mm, normp, b, w):
    d = mm.shape[1]
    dout = w.shape[1]
    return pl.pallas_call(
        _fuse_body,
        grid=(_NBP,),
        in_specs=[pl.BlockSpec((_RBP, d), lambda i: (i, 0)),
                  pl.BlockSpec((_RBP, d), lambda i: (_NBP + i, 0)),
                  pl.BlockSpec((_RBP, 1), lambda i: (i, 0)),
                  pl.BlockSpec((d,), lambda i: (0,)),
                  pl.BlockSpec((d, dout), lambda i: (0, 0))],
        out_specs=pl.BlockSpec((_RBP, dout), lambda i: (i, 0)),
        out_shape=jax.ShapeDtypeStruct((NPAD, dout), jnp.float32),
    )(mm, mm, normp, b, w)


def _final_body(ma_ref, mb_ref, n_ref, b_ref, o_ref):
    m = ma_ref[...] + mb_ref[...]
    o_ref[...] = m * n_ref[...] + b_ref[...][None, :]


def _final(mm, normp, b):
    d = mm.shape[1]
    return pl.pallas_call(
        _final_body,
        grid=(_NBP,),
        in_specs=[pl.BlockSpec((_RBP, d), lambda i: (i, 0)),
                  pl.BlockSpec((_RBP, d), lambda i: (_NBP + i, 0)),
                  pl.BlockSpec((_RBP, 1), lambda i: (i, 0)),
                  pl.BlockSpec((d,), lambda i: (0,))],
        out_specs=pl.BlockSpec((_RBP, d), lambda i: (i, 0)),
        out_shape=jax.ShapeDtypeStruct((NPAD, d), jnp.float32),
    )(mm, mm, normp, b)


def kernel(x, edge_index, norm, W0, b0, W1, b1, W2, b2):
    epw_real = E // NW
    pade = EPW - epw_real
    pad_s = jnp.broadcast_to(
        (jnp.arange(pade, dtype=jnp.int32) * 41) % N, (NW, pade))
    pad_d = jnp.broadcast_to(
        N + jnp.arange(pade, dtype=jnp.int32) % (NPAD - N), (NW, pade))
    src = jnp.concatenate(
        [edge_index[0].reshape(NW, epw_real), pad_s], axis=1
    ).reshape(NW, NCHUNK, K)
    dst = jnp.concatenate(
        [edge_index[1].reshape(NW, epw_real), pad_d], axis=1
    ).reshape(NW, NCHUNK, K)
    normp = jnp.pad(norm.reshape(N, 1), ((0, NPAD - N), (0, 0)))
    zero128 = jnp.zeros((ZR, D), jnp.float32)
    zero64 = jnp.zeros((ZR, CP), jnp.float32)
    W2p = W2 if CP == C else jnp.pad(W2, ((0, 0), (0, CP - C)))
    b2p = b2 if CP == C else jnp.pad(b2, (0, CP - C))

    agg_d = _make_agg(D)
    agg_c = _make_agg(CP)

    p0 = _proj(x, W0)
    m0 = agg_d(p0, src, dst, zero128)
    p1 = _fuse(m0, normp, b0, W1)
    m1 = agg_d(p1, src, dst, zero128)
    p2 = _fuse(m1, normp, b1, W2p)
    m2 = agg_c(p2, src, dst, zero64)
    out = _final(m2, normp, b2p)
    return out[:N, :C] if CP != C else out[:N]


# speedup vs baseline: 1.0309x; 1.0051x over previous
import functools

import jax
import jax.numpy as jnp
import numpy as np
from jax import lax
from jax.experimental import pallas as pl
from jax.experimental.pallas import tpu as pltpu
from jax.experimental.pallas import tpu_sc as plsc

N = 10000
E = 320000
D = 128
H = 128
C = 40
CP = 40

NW = 32
K = 128
NCHUNK = 80
GK = 8
NG = NCHUNK // GK
EP = NW * NCHUNK * K
EPW = EP // NW
NPAD = 10240
RPT = NPAD // 16
ZR = K


def _agg_body(p_hbm, src_hbm, dst_hbm, zero_hbm, out_hbm,
              accum, src_v, dst_v, buf0, buf1, gsem0, gsem1, isem):
    c = lax.axis_index("c")
    s = lax.axis_index("s")
    w = c * 16 + s
    base = s * RPT
    bufs = (buf0, buf1)
    sems = (gsem0, gsem1)

    pltpu.sync_copy(zero_hbm, buf0)
    for i in range(RPT // ZR):
        pltpu.async_copy(buf0, accum.at[pl.ds(base + i * ZR, ZR)], isem)
    pltpu.sync_copy(src_hbm.at[w, pl.ds(0, GK)], src_v.at[0])
    pltpu.sync_copy(dst_hbm.at[w, pl.ds(0, GK)], dst_v.at[0])
    for i in range(RPT // ZR):
        pltpu.make_async_copy(buf0, accum.at[pl.ds(base + i * ZR, ZR)],
                              isem).wait()
    plsc.subcore_barrier()

    pltpu.async_copy(p_hbm.at[src_v.at[0, 0]], buf0, gsem0)

    def group(g, carry):
        p = lax.rem(g, 2)
        q = 1 - p
        more = g + 1 < NG

        @pl.when(more)
        def _prefetch():
            pltpu.async_copy(src_hbm.at[w, pl.ds((g + 1) * GK, GK)],
                             src_v.at[q], isem)
            pltpu.async_copy(dst_hbm.at[w, pl.ds((g + 1) * GK, GK)],
                             dst_v.at[q], isem)

        for j in range(GK):
            b = j % 2
            nb = 1 - b
            if j + 1 < GK:
                pltpu.async_copy(p_hbm.at[src_v.at[p, j + 1]],
                                 bufs[nb], sems[nb])
            else:
                @pl.when(more)
                def _next_group():
                    pltpu.make_async_copy(
                        src_hbm.at[w, pl.ds((g + 1) * GK, GK)],
                        src_v.at[q], isem).wait()
                    pltpu.make_async_copy(
                        dst_hbm.at[w, pl.ds((g + 1) * GK, GK)],
                        dst_v.at[q], isem).wait()
                    pltpu.async_copy(p_hbm.at[src_v.at[q, 0]],
                                     bufs[nb], sems[nb])
            pltpu.make_async_copy(p_hbm.at[src_v.at[p, j]],
                                  bufs[b], sems[b]).wait()
            pltpu.sync_copy(bufs[b], accum.at[dst_v.at[p, j]], add=True)
        return carry

    lax.fori_loop(0, NG, group, 0)

    plsc.subcore_barrier()
    pltpu.sync_copy(accum.at[pl.ds(base, ZR)], buf0)
    for i in range(RPT // ZR):
        bo = bufs[i % 2]
        bn = bufs[(i + 1) % 2]
        if i + 1 < RPT // ZR:
            pltpu.async_copy(accum.at[pl.ds(base + (i + 1) * ZR, ZR)],
                             bn, isem)
        pltpu.sync_copy(bo, out_hbm.at[pl.ds(c * NPAD + base + i * ZR, ZR)])
        if i + 1 < RPT // ZR:
            pltpu.make_async_copy(accum.at[pl.ds(base + (i + 1) * ZR, ZR)],
                                  bn, isem).wait()


@functools.cache
def _make_agg(dw: int):
    mesh = plsc.VectorSubcoreMesh(core_axis_name="c", subcore_axis_name="s")
    params = None
    if dw != 128:
        params = pltpu.CompilerParams(use_tc_tiling_on_sc=False)
    return pl.kernel(
        _agg_body,
        out_type=jax.ShapeDtypeStruct((2 * NPAD, dw), jnp.float32),
        mesh=mesh,
        compiler_params=params,
        scratch_types=[
            pltpu.VMEM_SHARED((NPAD, dw), jnp.float32),
            pltpu.VMEM((2, GK, K), jnp.int32),
            pltpu.VMEM((2, GK, K), jnp.int32),
            pltpu.VMEM((K, dw), jnp.float32),
            pltpu.VMEM((K, dw), jnp.float32),
            pltpu.SemaphoreType.DMA,
            pltpu.SemaphoreType.DMA,
            pltpu.SemaphoreType.DMA,
        ],
    )


_RB = 1000
_RBP = 1024
_NBP = NPAD // _RBP


def _proj_body(x_ref, w_ref, o_ref):
    o_ref[...] = jnp.dot(x_ref[...], w_ref[...],
                         preferred_element_type=jnp.float32)


def _proj(x, w):
    n, d = x.shape
    dout = w.shape[1]
    return pl.pallas_call(
        _proj_body,
        grid=(n // _RB,),
        in_specs=[pl.BlockSpec((_RB, d), lambda i: (i, 0)),
                  pl.BlockSpec((d, dout), lambda i: (0, 0))],
        out_specs=pl.BlockSpec((_RB, dout), lambda i: (i, 0)),
        out_shape=jax.ShapeDtypeStruct((n, dout), jnp.float32),
    )(x, w)


def _fuse_body(ma_ref, mb_ref, n_ref, b_ref, w_ref, o_ref):
    m = ma_ref[...] + mb_ref[...]
    h = jnp.maximum(m * n_ref[...] + b_ref[...][None, :], 0.0)
    o_ref[...] = jnp.dot(h, w_ref[...], preferred_element_type=jnp.float32)


def _fuse(mm, normp, b, w):
    d = mm.shape[1]
    dout = w.shape[1]
    return pl.pallas_call(
        _fuse_body,
        grid=(_NBP,),
        in_specs=[pl.BlockSpec((_RBP, d), lambda i: (i, 0)),
                  pl.BlockSpec((_RBP, d), lambda i: (_NBP + i, 0)),
                  pl.BlockSpec((_RBP, 1), lambda i: (i, 0)),
                  pl.BlockSpec((d,), lambda i: (0,)),
                  pl.BlockSpec((d, dout), lambda i: (0, 0))],
        out_specs=pl.BlockSpec((_RBP, dout), lambda i: (i, 0)),
        out_shape=jax.ShapeDtypeStruct((NPAD, dout), jnp.float32),
    )(mm, mm, normp, b, w)


def _final_body(ma_ref, mb_ref, n_ref, b_ref, o_ref):
    m = ma_ref[...] + mb_ref[...]
    o_ref[...] = m * n_ref[...] + b_ref[...][None, :]


def _final(mm, normp, b):
    d = mm.shape[1]
    return pl.pallas_call(
        _final_body,
        grid=(_NBP,),
        in_specs=[pl.BlockSpec((_RBP, d), lambda i: (i, 0)),
                  pl.BlockSpec((_RBP, d), lambda i: (_NBP + i, 0)),
                  pl.BlockSpec((_RBP, 1), lambda i: (i, 0)),
                  pl.BlockSpec((d,), lambda i: (0,))],
        out_specs=pl.BlockSpec((_RBP, d), lambda i: (i, 0)),
        out_shape=jax.ShapeDtypeStruct((NPAD, d), jnp.float32),
    )(mm, mm, normp, b)


def kernel(x, edge_index, norm, W0, b0, W1, b1, W2, b2):
    epw_real = E // NW
    pade = EPW - epw_real
    pad_s = jnp.asarray(np.broadcast_to(
        (np.arange(pade, dtype=np.int32) * 41) % N, (NW, pade)))
    pad_d = jnp.asarray(np.broadcast_to(
        N + np.arange(pade, dtype=np.int32) % (NPAD - N), (NW, pade)))
    src = jnp.concatenate(
        [edge_index[0].reshape(NW, epw_real), pad_s], axis=1
    ).reshape(NW, NCHUNK, K)
    dst = jnp.concatenate(
        [edge_index[1].reshape(NW, epw_real), pad_d], axis=1
    ).reshape(NW, NCHUNK, K)
    normp = jnp.pad(norm.reshape(N, 1), ((0, NPAD - N), (0, 0)))
    zero128 = jnp.zeros((ZR, D), jnp.float32)
    zero64 = jnp.zeros((ZR, CP), jnp.float32)
    W2p = W2 if CP == C else jnp.pad(W2, ((0, 0), (0, CP - C)))
    b2p = b2 if CP == C else jnp.pad(b2, (0, CP - C))

    agg_d = _make_agg(D)
    agg_c = _make_agg(CP)

    p0 = _proj(x, W0)
    m0 = agg_d(p0, src, dst, zero128)
    p1 = _fuse(m0, normp, b0, W1)
    m1 = agg_d(p1, src, dst, zero128)
    p2 = _fuse(m1, normp, b1, W2p)
    m2 = agg_c(p2, src, dst, zero64)
    out = _final(m2, normp, b2p)
    return out[:N, :C] if CP != C else out[:N]
